# baseline (speedup 1.0000x reference)
import numpy as np
import jax
import jax.numpy as jnp
from jax import lax
from jax.experimental import pallas as pl
from jax.experimental.pallas import tpu as pltpu
from jax.experimental.pallas import tpu_sc as plsc

_C = 128
_K = 16
_N = 10000
_NC, _NS, _L = 2, 16, 16
_NW = _NC * _NS
_NPAD = 10240
_P = _NPAD // _NW
_G = 4
_NG = _P // _G
_NG_LAST = (_N - (_NW - 1) * _P) // _G
_BN = 256


def _proj_body(spa_ref, spe_ref, w_ref, b_ref, q_ref, t_ref):
    x = jnp.concatenate([spa_ref[...], spe_ref[...]], axis=0)
    y = lax.dot_general(x, w_ref[...], (((0,), (0,)), ((), ())),
                        preferred_element_type=jnp.float32)
    y = y + b_ref[...]
    q_ref[...] = y[:, : 2 * _C]
    t_ref[...] = y[:, 2 * _C:]


def _epi_body(spa_ref, spe_ref, ctx_ref, out_ref):
    ctx_t = jnp.transpose(ctx_ref[...], (1, 0))
    out_ref[...] = jnp.concatenate([spa_ref[...], spe_ref[...]], axis=0) + ctx_t


def _lane_bcast(v, k):
    idx = jnp.full((16,), k, jnp.int32)
    return v.at[idx].get(mode="promise_in_bounds")


def _sc_attention(q_hbm, t_hbm, idx_hbm, out_hbm,
                  idx_v, rows0, rows1, q0, q1, out0, out1, sa_v, se_v,
                  sem_r0, sem_r1, sem_q0, sem_q1, sem_o0, sem_o1):
    wid = lax.axis_index("s") * _NC + lax.axis_index("c")
    base = wid * _P
    iota16 = lax.iota(jnp.int32, 16)
    last = wid == _NW - 1
    ng = jnp.where(last, _NG_LAST, _NG)

    @pl.when(jnp.logical_not(last))
    def _():
        pltpu.sync_copy(idx_hbm.at[pl.ds(base * _K, _P * _K)], idx_v)

    @pl.when(last)
    def _():
        pltpu.sync_copy(
            idx_hbm.at[pl.ds(base * _K, _NG_LAST * _G * _K)],
            idx_v.at[pl.ds(0, _NG_LAST * _G * _K)])

    def _start(g, rows_v, q_v, sem_r, sem_q):
        pltpu.async_copy(t_hbm.at[idx_v.at[pl.ds(g * (_G * _K), _G * _K)]],
                         rows_v, sem_r)
        pltpu.async_copy(q_hbm.at[pl.ds(base + g * _G, _G)], q_v, sem_q)

    def _wait_in(rows_v, q_v, sem_r, sem_q):
        pltpu.make_async_copy(t_hbm.at[idx_v.at[pl.ds(0, _G * _K)]],
                              rows_v, sem_r).wait()
        pltpu.make_async_copy(q_hbm.at[pl.ds(0, _G)], q_v, sem_q).wait()

    def _wait_out(out_v, sem_o):
        pltpu.make_async_copy(out_v, out_hbm.at[pl.ds(0, _G)], sem_o).wait()

    def _compute(rows_v, q_v, out_v):
        for p in range(_G):
            base_r = p * _K
            qa = [q_v[p, pl.ds(j * 16, 16)] for j in range(8)]
            qe = [q_v[p, pl.ds(_C + j * 16, 16)] for j in range(8)]

            def _dots(k, _):
                r = base_r + k
                acca = jnp.zeros((16,), jnp.float32)
                acce = jnp.zeros((16,), jnp.float32)
                for j in range(8):
                    acca = acca + qa[j] * rows_v[r, pl.ds(j * 16, 16)]
                    acce = acce + qe[j] * rows_v[r, pl.ds(2 * _C + j * 16, 16)]
                flat = iota16 * 16 + k
                plsc.store_scatter(sa_v, [flat], acca)
                plsc.store_scatter(se_v, [flat], acce)
                return 0

            lax.fori_loop(0, _K, _dots, 0, unroll=4)

            sa = sa_v[pl.ds(0, 16)]
            se = se_v[pl.ds(0, 16)]
            for i in range(1, 16):
                sa = sa + sa_v[pl.ds(i * 16, 16)]
                se = se + se_v[pl.ds(i * 16, 16)]

            aa = jnp.exp(sa - jnp.max(sa))
            aa = aa / jnp.sum(aa)
            ae = jnp.exp(se - jnp.max(se))
            ae = ae / jnp.sum(ae)

            def _ctx(k, accs):
                r = base_r + k
                ak = _lane_bcast(aa, k)
                ek = _lane_bcast(ae, k)
                va = tuple(accs[j] + ak * rows_v[r, pl.ds(_C + j * 16, 16)]
                           for j in range(8))
                ve = tuple(accs[8 + j] + ek * rows_v[r, pl.ds(3 * _C + j * 16, 16)]
                           for j in range(8))
                return va + ve

            accs = lax.fori_loop(
                0, _K, _ctx,
                tuple(jnp.zeros((16,), jnp.float32) for _ in range(16)),
                unroll=4)
            for j in range(8):
                out_v[p, pl.ds(j * 16, 16)] = accs[j]
                out_v[p, pl.ds(_C + j * 16, 16)] = accs[8 + j]

    _start(0, rows0, q0, sem_r0, sem_q0)

    @pl.loop(0, ng, step=2)
    def _pair(gg):
        _start(gg + 1, rows1, q1, sem_r1, sem_q1)
        _wait_in(rows0, q0, sem_r0, sem_q0)

        @pl.when(gg > 0)
        def _():
            _wait_out(out0, sem_o0)

        _compute(rows0, q0, out0)
        pltpu.async_copy(out0, out_hbm.at[pl.ds(base + gg * _G, _G)], sem_o0)

        @pl.when(gg + 2 < ng)
        def _():
            _start(gg + 2, rows0, q0, sem_r0, sem_q0)

        _wait_in(rows1, q1, sem_r1, sem_q1)

        @pl.when(gg > 0)
        def _():
            _wait_out(out1, sem_o1)

        _compute(rows1, q1, out1)
        pltpu.async_copy(out1, out_hbm.at[pl.ds(base + (gg + 1) * _G, _G)],
                         sem_o1)

    _wait_out(out0, sem_o0)
    _wait_out(out1, sem_o1)


def kernel(spa, spe, neighbor_indices,
           Wq_spa, Wk_spa, Wv_spa, bv_spa,
           Wq_spe, Wk_spe, Wv_spe, bv_spe):
    f32 = jnp.float32
    spa2 = spa[0]
    spe2 = spe[0]
    scale = np.float32(1.0 / np.sqrt(_C))
    Z = jnp.zeros((_C, _C), f32)
    W = jnp.concatenate([
        jnp.concatenate([Wq_spa * scale, Z, Z, Z, Wk_spe, Wv_spe], axis=1),
        jnp.concatenate([Z, Wq_spe * scale, Wk_spa, Wv_spa, Z, Z], axis=1),
    ], axis=0)
    zc = jnp.zeros((_C,), f32)
    b = jnp.concatenate([zc, zc, zc, bv_spa, zc, bv_spe])[None, :]

    q, t = pl.pallas_call(
        _proj_body,
        grid=(_NPAD // _BN,),
        in_specs=[
            pl.BlockSpec((_C, _BN), lambda i: (0, i)),
            pl.BlockSpec((_C, _BN), lambda i: (0, i)),
            pl.BlockSpec((2 * _C, 6 * _C), lambda i: (0, 0)),
            pl.BlockSpec((1, 6 * _C), lambda i: (0, 0)),
        ],
        out_specs=[
            pl.BlockSpec((_BN, 2 * _C), lambda i: (i, 0)),
            pl.BlockSpec((_BN, 4 * _C), lambda i: (i, 0)),
        ],
        out_shape=[
            jax.ShapeDtypeStruct((_NPAD, 2 * _C), f32),
            jax.ShapeDtypeStruct((_NPAD, 4 * _C), f32),
        ],
    )(spa2, spe2, W, b)

    idx_flat = neighbor_indices[0].astype(jnp.int32).reshape(-1)

    sc_att = pl.kernel(
        _sc_attention,
        out_type=jax.ShapeDtypeStruct((_NPAD, 2 * _C), f32),
        mesh=plsc.VectorSubcoreMesh(core_axis_name="c", subcore_axis_name="s"),
        compiler_params=pltpu.CompilerParams(needs_layout_passes=False),
        scratch_types=[
            pltpu.VMEM((_P * _K,), jnp.int32),
            pltpu.VMEM((_G * _K, 4 * _C), f32),
            pltpu.VMEM((_G * _K, 4 * _C), f32),
            pltpu.VMEM((_G, 2 * _C), f32),
            pltpu.VMEM((_G, 2 * _C), f32),
            pltpu.VMEM((_G, 2 * _C), f32),
            pltpu.VMEM((_G, 2 * _C), f32),
            pltpu.VMEM((_L * _L,), f32),
            pltpu.VMEM((_L * _L,), f32),
            pltpu.SemaphoreType.DMA,
            pltpu.SemaphoreType.DMA,
            pltpu.SemaphoreType.DMA,
            pltpu.SemaphoreType.DMA,
            pltpu.SemaphoreType.DMA,
            pltpu.SemaphoreType.DMA,
        ],
    )
    ctx = sc_att(q, t, idx_flat)

    out = pl.pallas_call(
        _epi_body,
        grid=(_NPAD // _BN,),
        in_specs=[
            pl.BlockSpec((_C, _BN), lambda i: (0, i)),
            pl.BlockSpec((_C, _BN), lambda i: (0, i)),
            pl.BlockSpec((_BN, 2 * _C), lambda i: (i, 0)),
        ],
        out_specs=pl.BlockSpec((2 * _C, _BN), lambda i: (0, i)),
        out_shape=jax.ShapeDtypeStruct((2 * _C, _N), f32),
    )(spa2, spe2, ctx)

    return out[None]
